# baseline (speedup 1.0000x reference)
import jax
import jax.numpy as jnp
from jax import lax
from jax.experimental import pallas as pl
from jax.experimental.pallas import tpu as pltpu
from jax.experimental.pallas import tpu_sc as plsc

NC = 2
NS = 16
NW = NC * NS
LANES = 16
SUB = 8
LANE = 128


def _make_gather(Dn, Cn, Bn):
    total = Bn * Dn
    n_per = total // NW
    period = SUB * Dn
    chunk = period
    n_chunks = n_per // chunk
    log2c = Cn.bit_length() - 1
    log2l = LANE.bit_length() - 1
    assert n_per % chunk == 0 and Cn == (1 << log2c) and n_chunks >= 3
    row_stride = Cn * SUB
    lane_mul = (1 << 10) - LANE

    mesh = plsc.VectorSubcoreMesh(core_axis_name="c", subcore_axis_name="s")

    depth = 4
    nbuf = 6

    def body(w_hbm, idx_hbm, out_hbm, *rest):
        ivs = rest[0:nbuf]
        ovs = rest[nbuf:2 * nbuf]
        isems = rest[2 * nbuf:3 * nbuf]
        gsems = rest[3 * nbuf:3 * nbuf + depth]
        ssems = rest[3 * nbuf + depth:3 * nbuf + depth + nbuf]
        wid = lax.axis_index("s") * NC + lax.axis_index("c")
        base = wid * n_per

        iota = lax.iota(jnp.int32, LANES)
        constvec = ((iota >> 3) * row_stride) + ((iota & 7) * LANE)

        def idx_dma(k):
            return pltpu.make_async_copy(
                idx_hbm.at[pl.ds(base + k * chunk, chunk)],
                ivs[k % nbuf], isems[k % nbuf])

        def store_dma(k):
            return pltpu.make_async_copy(
                ovs[k % nbuf], out_hbm.at[pl.ds(base + k * chunk, chunk)],
                ssems[k % nbuf])

        def addr(k):
            idx_b = ivs[k % nbuf]

            def addr_body(jj, _):
                j = jj * LANES
                dbase = ((j >> 10) << log2l) + (j & (LANE - 1))
                sl = pl.ds(j, LANES)
                iv = idx_b[sl]
                pw = (iv + (iv >> log2l) * lane_mul
                      + ((dbase >> 3) * row_stride + constvec))
                idx_b[sl] = (pw & 0x3FFF) + wid * 16384
                return _

            lax.fori_loop(0, chunk // LANES, addr_body, None, unroll=8)

        def gather_dma(k):
            return pltpu.make_async_copy(
                w_hbm.at[ivs[k % nbuf]], ovs[k % nbuf], gsems[k % depth])

        for j in range(min(depth + 1, n_chunks)):
            idx_dma(j).start()
        for j in range(min(depth, n_chunks)):
            idx_dma(j).wait()
            addr(j)
            gather_dma(j).start()
        for k in range(n_chunks):
            gather_dma(k).wait()
            store_dma(k).start()
            nk = k + depth
            if nk < n_chunks:
                if nk + 1 < n_chunks:
                    idx_dma(nk + 1).start()
                idx_dma(nk).wait()
                addr(nk)
                if nk - nbuf >= 0:
                    store_dma(nk - nbuf).wait()
                gather_dma(nk).start()
        for k in range(max(0, n_chunks - nbuf), n_chunks):
            if k + depth >= nbuf:
                store_dma(k).wait()

    return pl.kernel(
        body,
        out_type=jax.ShapeDtypeStruct((total,), jnp.float32),
        mesh=mesh,
        scratch_types=(
            [pltpu.VMEM((chunk,), jnp.int32) for _ in range(6)]
            + [pltpu.VMEM((chunk,), jnp.float32) for _ in range(6)]
            + [pltpu.SemaphoreType.DMA for _ in range(6 + 4 + 6)]
        ),
    )


def kernel(weights, indices):
    Dn, Cn = weights.shape
    Bn, _ = indices.shape
    wp = weights.reshape(Dn // SUB, SUB, Cn // LANE, LANE)
    wp = wp.transpose(0, 2, 1, 3).reshape(-1)
    ip = indices.reshape(Bn // SUB, SUB, Dn // LANE, LANE)
    ip = ip.transpose(0, 2, 1, 3).reshape(-1)
    out_phys = _make_gather(Dn, Cn, Bn)(wp, ip)
    out = out_phys.reshape(Bn // SUB, Dn // LANE, SUB, LANE)
    out = out.transpose(0, 2, 1, 3).reshape(Bn, Dn)
    return out


# speedup vs baseline: 1.1311x; 1.1311x over previous
import jax
import jax.numpy as jnp
from jax import lax
from jax.experimental import pallas as pl
from jax.experimental.pallas import tpu as pltpu
from jax.experimental.pallas import tpu_sc as plsc

NC = 2
NS = 16
NW = NC * NS
LANES = 16
SUB = 8
LANE = 128


def _make_gather(Dn, Cn, Bn):
    total = Bn * Dn
    n_per = total // NW
    period = SUB * Dn
    chunk = 2 * period
    n_chunks = n_per // chunk
    log2c = Cn.bit_length() - 1
    log2l = LANE.bit_length() - 1
    assert n_per % chunk == 0 and Cn == (1 << log2c) and n_chunks >= 3
    row_stride = Cn * SUB
    lane_mul = (1 << 10) - LANE

    mesh = plsc.VectorSubcoreMesh(core_axis_name="c", subcore_axis_name="s")

    depth = 2
    nbuf = 3

    def body(w_hbm, idx_hbm, out_hbm, *rest):
        ivs = rest[0:nbuf]
        ovs = rest[nbuf:2 * nbuf]
        isems = rest[2 * nbuf:3 * nbuf]
        gsems = rest[3 * nbuf:3 * nbuf + depth]
        ssems = rest[3 * nbuf + depth:3 * nbuf + depth + nbuf]
        wid = lax.axis_index("s") * NC + lax.axis_index("c")
        base = wid * n_per

        iota = lax.iota(jnp.int32, LANES)
        constvec = ((iota >> 3) * row_stride) + ((iota & 7) * LANE)

        def idx_dma(k):
            return pltpu.make_async_copy(
                idx_hbm.at[pl.ds(base + k * chunk, chunk)],
                ivs[k % nbuf], isems[k % nbuf])

        def store_dma(k):
            return pltpu.make_async_copy(
                ovs[k % nbuf], out_hbm.at[pl.ds(base + k * chunk, chunk)],
                ssems[k % nbuf])

        def addr(k):
            idx_b = ivs[k % nbuf]

            def addr_body(jj, _):
                j = jj * LANES
                jp = j & (period - 1)
                dbase = ((jp >> 10) << log2l) + (jp & (LANE - 1))
                sl = pl.ds(j, LANES)
                iv = idx_b[sl]
                pw = (iv + (iv >> log2l) * lane_mul
                      + ((dbase >> 3) * row_stride + constvec))
                idx_b[sl] = pw
                return _

            lax.fori_loop(0, chunk // LANES, addr_body, None, unroll=8)

        def gather_dma(k):
            return pltpu.make_async_copy(
                w_hbm.at[ivs[k % nbuf]], ovs[k % nbuf], gsems[k % depth])

        for j in range(min(depth + 1, n_chunks)):
            idx_dma(j).start()
        for j in range(min(depth, n_chunks)):
            idx_dma(j).wait()
            addr(j)
            gather_dma(j).start()
        for k in range(n_chunks):
            gather_dma(k).wait()
            store_dma(k).start()
            nk = k + depth
            if nk < n_chunks:
                if nk + 1 < n_chunks:
                    idx_dma(nk + 1).start()
                idx_dma(nk).wait()
                addr(nk)
                if nk - nbuf >= 0:
                    store_dma(nk - nbuf).wait()
                gather_dma(nk).start()
        for k in range(max(0, n_chunks - nbuf), n_chunks):
            if k + depth >= nbuf:
                store_dma(k).wait()

    return pl.kernel(
        body,
        out_type=jax.ShapeDtypeStruct((total,), jnp.float32),
        mesh=mesh,
        scratch_types=(
            [pltpu.VMEM((chunk,), jnp.int32) for _ in range(3)]
            + [pltpu.VMEM((chunk,), jnp.float32) for _ in range(3)]
            + [pltpu.SemaphoreType.DMA for _ in range(3 + 2 + 3)]
        ),
    )


def kernel(weights, indices):
    Dn, Cn = weights.shape
    Bn, _ = indices.shape
    wp = weights.reshape(Dn // SUB, SUB, Cn // LANE, LANE)
    wp = wp.transpose(0, 2, 1, 3).reshape(-1)
    ip = indices.reshape(Bn // SUB, SUB, Dn // LANE, LANE)
    ip = ip.transpose(0, 2, 1, 3).reshape(-1)
    out_phys = _make_gather(Dn, Cn, Bn)(wp, ip)
    out = out_phys.reshape(Bn // SUB, Dn // LANE, SUB, LANE)
    out = out.transpose(0, 2, 1, 3).reshape(Bn, Dn)
    return out


# speedup vs baseline: 1.1428x; 1.0103x over previous
import jax
import jax.numpy as jnp
from jax import lax
from jax.experimental import pallas as pl
from jax.experimental.pallas import tpu as pltpu
from jax.experimental.pallas import tpu_sc as plsc

NC = 2
NS = 16
NW = NC * NS
LANES = 16
SUB = 8
LANE = 128


def _make_gather(Dn, Cn, Bn):
    total = Bn * Dn
    n_per = total // NW
    period = SUB * Dn
    chunk = period
    n_chunks = n_per // chunk
    log2c = Cn.bit_length() - 1
    log2l = LANE.bit_length() - 1
    assert n_per % chunk == 0 and Cn == (1 << log2c) and n_chunks >= 3
    row_stride = Cn * SUB
    lane_mul = (1 << 10) - LANE

    mesh = plsc.VectorSubcoreMesh(core_axis_name="c", subcore_axis_name="s")

    depth = 4
    nbuf = 6

    def body(w_hbm, idx_hbm, out_hbm, *rest):
        ivs = rest[0:nbuf]
        ovs = rest[nbuf:2 * nbuf]
        isems = rest[2 * nbuf:3 * nbuf]
        gsems = rest[3 * nbuf:3 * nbuf + depth]
        ssems = rest[3 * nbuf + depth:3 * nbuf + depth + nbuf]
        wid = lax.axis_index("s") * NC + lax.axis_index("c")
        base = wid * n_per

        iota = lax.iota(jnp.int32, LANES)
        constvec = ((iota >> 3) * row_stride) + ((iota & 7) * LANE)

        def idx_dma(k):
            return pltpu.make_async_copy(
                idx_hbm.at[pl.ds(base + k * chunk, chunk)],
                ivs[k % nbuf], isems[k % nbuf])

        def store_dma(k):
            return pltpu.make_async_copy(
                ovs[k % nbuf], out_hbm.at[pl.ds(base + k * chunk, chunk)],
                ssems[k % nbuf])

        def addr(k):
            idx_b = ivs[k % nbuf]

            def addr_body(jj, _):
                j = jj * LANES
                jp = j & (period - 1)
                dbase = ((jp >> 10) << log2l) + (jp & (LANE - 1))
                sl = pl.ds(j, LANES)
                iv = idx_b[sl]
                pw = (iv + (iv >> log2l) * lane_mul
                      + ((dbase >> 3) * row_stride + constvec))
                idx_b[sl] = pw
                return _

            lax.fori_loop(0, chunk // LANES, addr_body, None, unroll=8)

        def gather_dma(k):
            return pltpu.make_async_copy(
                w_hbm.at[ivs[k % nbuf]], ovs[k % nbuf], gsems[k % depth])

        for j in range(min(depth + 1, n_chunks)):
            idx_dma(j).start()
        for j in range(min(depth, n_chunks)):
            idx_dma(j).wait()
            addr(j)
            gather_dma(j).start()
        for k in range(n_chunks):
            gather_dma(k).wait()
            store_dma(k).start()
            nk = k + depth
            if nk < n_chunks:
                if nk + 1 < n_chunks:
                    idx_dma(nk + 1).start()
                idx_dma(nk).wait()
                addr(nk)
                if nk - nbuf >= 0:
                    store_dma(nk - nbuf).wait()
                gather_dma(nk).start()
        for k in range(max(0, n_chunks - nbuf), n_chunks):
            if k + depth >= nbuf:
                store_dma(k).wait()

    return pl.kernel(
        body,
        out_type=jax.ShapeDtypeStruct((total,), jnp.float32),
        mesh=mesh,
        scratch_types=(
            [pltpu.VMEM((chunk,), jnp.int32) for _ in range(6)]
            + [pltpu.VMEM((chunk,), jnp.float32) for _ in range(6)]
            + [pltpu.SemaphoreType.DMA for _ in range(6 + 4 + 6)]
        ),
    )


def kernel(weights, indices):
    Dn, Cn = weights.shape
    Bn, _ = indices.shape
    wp = weights.reshape(Dn // SUB, SUB, Cn // LANE, LANE)
    wp = wp.transpose(0, 2, 1, 3).reshape(-1)
    ip = indices.reshape(Bn // SUB, SUB, Dn // LANE, LANE)
    ip = ip.transpose(0, 2, 1, 3).reshape(-1)
    out_phys = _make_gather(Dn, Cn, Bn)(wp, ip)
    out = out_phys.reshape(Bn // SUB, Dn // LANE, SUB, LANE)
    out = out.transpose(0, 2, 1, 3).reshape(Bn, Dn)
    return out


# speedup vs baseline: 1.1435x; 1.0006x over previous
import jax
import jax.numpy as jnp
from jax import lax
from jax.experimental import pallas as pl
from jax.experimental.pallas import tpu as pltpu
from jax.experimental.pallas import tpu_sc as plsc

NC = 2
NS = 16
NW = NC * NS
LANES = 16
SUB = 8
LANE = 128


def _make_gather(Dn, Cn, Bn):
    total = Bn * Dn
    n_per = total // NW
    period = SUB * Dn
    chunk = period
    n_chunks = n_per // chunk
    log2c = Cn.bit_length() - 1
    log2l = LANE.bit_length() - 1
    tile_words = SUB * LANE
    log2t = tile_words.bit_length() - 1
    assert n_per % chunk == 0 and Cn == (1 << log2c) and n_chunks >= 3
    assert Dn % LANE == 0 and Cn % LANE == 0 and Bn % SUB == 0
    row_stride = Cn * SUB
    lane_mul = tile_words - LANE

    mesh = plsc.VectorSubcoreMesh(core_axis_name="c", subcore_axis_name="s")

    depth = 4
    nbuf = 6

    def body(w_hbm, idx_hbm, out_hbm, *rest):
        ivs = rest[0:nbuf]
        ovs = rest[nbuf:2 * nbuf]
        isems = rest[2 * nbuf:3 * nbuf]
        gsems = rest[3 * nbuf:3 * nbuf + depth]
        ssems = rest[3 * nbuf + depth:3 * nbuf + depth + nbuf]
        wid = lax.axis_index("s") * NC + lax.axis_index("c")
        base = wid * n_per

        iota = lax.iota(jnp.int32, LANES)
        constvec = ((iota >> 3) * row_stride) + ((iota & 7) * LANE)

        def idx_dma(k):
            return pltpu.make_async_copy(
                idx_hbm.at[pl.ds(base + k * chunk, chunk)],
                ivs[k % nbuf], isems[k % nbuf])

        def store_dma(k):
            return pltpu.make_async_copy(
                ovs[k % nbuf], out_hbm.at[pl.ds(base + k * chunk, chunk)],
                ssems[k % nbuf])

        def addr(k):
            idx_b = ivs[k % nbuf]

            def addr_body(jj, _):
                j = jj * LANES
                jp = j & (period - 1)
                dbase = ((jp >> log2t) << log2l) + (jp & (LANE - 1))
                sl = pl.ds(j, LANES)
                iv = idx_b[sl]
                pw = (iv + (iv >> log2l) * lane_mul
                      + ((dbase >> 3) * row_stride + constvec))
                idx_b[sl] = pw
                return _

            lax.fori_loop(0, chunk // LANES, addr_body, None, unroll=8)

        def gather_dma(k):
            return pltpu.make_async_copy(
                w_hbm.at[ivs[k % nbuf]], ovs[k % nbuf], gsems[k % depth])

        for j in range(min(depth + 1, n_chunks)):
            idx_dma(j).start()
        for j in range(min(depth, n_chunks)):
            idx_dma(j).wait()
            addr(j)
            gather_dma(j).start()
        for k in range(n_chunks):
            gather_dma(k).wait()
            store_dma(k).start()
            nk = k + depth
            if nk < n_chunks:
                if nk + 1 < n_chunks:
                    idx_dma(nk + 1).start()
                idx_dma(nk).wait()
                addr(nk)
                if nk - nbuf >= 0:
                    store_dma(nk - nbuf).wait()
                gather_dma(nk).start()
        for k in range(max(0, n_chunks - nbuf), n_chunks):
            if k + depth >= nbuf:
                store_dma(k).wait()

    return pl.kernel(
        body,
        out_type=jax.ShapeDtypeStruct((total,), jnp.float32),
        mesh=mesh,
        scratch_types=(
            [pltpu.VMEM((chunk,), jnp.int32) for _ in range(nbuf)]
            + [pltpu.VMEM((chunk,), jnp.float32) for _ in range(nbuf)]
            + [pltpu.SemaphoreType.DMA for _ in range(2 * nbuf + depth)]
        ),
    )


def kernel(weights, indices):
    Dn, Cn = weights.shape
    Bn, _ = indices.shape
    wp = weights.reshape(Dn // SUB, SUB, Cn // LANE, LANE)
    wp = wp.transpose(0, 2, 1, 3).reshape(-1)
    ip = indices.reshape(Bn // SUB, SUB, Dn // LANE, LANE)
    ip = ip.transpose(0, 2, 1, 3).reshape(-1)
    out_phys = _make_gather(Dn, Cn, Bn)(wp, ip)
    out = out_phys.reshape(Bn // SUB, Dn // LANE, SUB, LANE)
    out = out.transpose(0, 2, 1, 3).reshape(Bn, Dn)
    return out
